# baseline (speedup 1.0000x reference)
import functools

import jax
import jax.numpy as jnp
from jax import lax
from jax.experimental import pallas as pl
from jax.experimental.pallas import tpu as pltpu
from jax.experimental.pallas import tpu_sc as plsc

_N = 14
_R = 4
_Q = 2 * _N - 1
_PAD = _N - 1 - _R
_NC = 2
_NS = 16
_NW = _NC * _NS
_NROW = _N * _N * _N
_ROWPAD = ((_NROW + 15) // 16) * 16
_NCHUNK = _ROWPAD // 16
_QFPAD = ((_Q ** 3 + 7) // 8) * 8
_C0 = (_N - 1) * (_Q * _Q + _Q + 1)
_G = 16
_NGRP = 6


def _row_base(q):
    i = q // (_N * _N)
    rem = q - i * (_N * _N)
    j = rem // _N
    k = rem - j * _N
    return _C0 - (i * (_Q * _Q) + j * _Q + k)


def _fill_body(qf_hbm, cc_hbm, out_hbm, qf_v, cc_v, row_v, sems):
    wid = lax.axis_index("s") * _NC + lax.axis_index("c")
    pltpu.sync_copy(qf_hbm, qf_v)
    pltpu.sync_copy(cc_hbm, cc_v)

    start = wid * 86 - jnp.maximum(wid - 24, 0)
    count = jnp.where(wid < 24, 86, 85)

    def build_group(gstart, parity):
        bases = [_row_base(gstart + r) for r in range(_G)]

        @plsc.parallel_loop(0, _NCHUNK, unroll=4)
        def chunk_step(t2):
            cc = cc_v[pl.ds(t2 * 16, 16)]
            idxs = [cc + jnp.int32(bases[r]) for r in range(_G)]
            vals = [plsc.load_gather(qf_v, [idxs[r]]) for r in range(_G)]
            for r in range(_G):
                row_v[parity, r, pl.ds(t2 * 16, 16)] = vals[r]

    for g in range(_NGRP):
        parity = g % 2
        gstart = jnp.minimum(start + g * _G, start + count - _G)
        if g >= 2:
            pltpu.make_async_copy(
                row_v.at[parity, :, pl.ds(0, _NROW)],
                out_hbm.at[pl.ds(0, _G)], sems.at[parity]).wait()
        build_group(gstart, parity)
        pltpu.async_copy(
            row_v.at[parity, :, pl.ds(0, _NROW)],
            out_hbm.at[pl.ds(gstart, _G)], sems.at[parity])

    for parity in range(2):
        pltpu.make_async_copy(
            row_v.at[parity, :, pl.ds(0, _NROW)],
            out_hbm.at[pl.ds(0, _G)], sems.at[parity]).wait()


@jax.jit
def _sc_fill(qf, colcode):
    mesh = plsc.VectorSubcoreMesh(
        core_axis_name="c", subcore_axis_name="s",
        num_cores=_NC, num_subcores=_NS,
    )
    return pl.kernel(
        _fill_body,
        out_type=jax.ShapeDtypeStruct((_NROW, _NROW), jnp.float32),
        mesh=mesh,
        scratch_types=[
            pltpu.VMEM((_QFPAD,), jnp.float32),
            pltpu.VMEM((_ROWPAD,), jnp.int32),
            pltpu.VMEM((2, _G, _ROWPAD), jnp.float32),
            pltpu.SemaphoreType.DMA((2,)),
        ],
        compiler_params=pltpu.CompilerParams(
            use_tc_tiling_on_sc=False, needs_layout_passes=False),
    )(qf, colcode)


def kernel(biases, feat_H, feat_W, feat_D):
    del feat_H, feat_W, feat_D
    q = jnp.pad(biases.astype(jnp.float32), _PAD)
    qf = jnp.zeros((_QFPAD,), jnp.float32).at[: _Q ** 3].set(q.reshape(-1))
    abc = jnp.arange(_NROW, dtype=jnp.int32)
    a, b, c = abc // (_N * _N), (abc // _N) % _N, abc % _N
    colcode = jnp.zeros((_ROWPAD,), jnp.int32).at[:_NROW].set(
        a * (_Q * _Q) + b * _Q + c)
    return _sc_fill(qf, colcode)


# speedup vs baseline: 1.0440x; 1.0440x over previous
import functools

import jax
import jax.numpy as jnp
from jax import lax
from jax.experimental import pallas as pl
from jax.experimental.pallas import tpu as pltpu
from jax.experimental.pallas import tpu_sc as plsc

_N = 14
_R = 4
_Q = 2 * _N - 1
_PAD = _N - 1 - _R
_NC = 2
_NS = 16
_NW = _NC * _NS
_NROW = _N * _N * _N
_ROWPAD = ((_NROW + 15) // 16) * 16
_NCHUNK = _ROWPAD // 16
_QFPAD = ((_Q ** 3 + 7) // 8) * 8
_C0 = (_N - 1) * (_Q * _Q + _Q + 1)
_G = 16
_NGRP = 6


def _row_base(q):
    i = q // (_N * _N)
    rem = q - i * (_N * _N)
    j = rem // _N
    k = rem - j * _N
    return _C0 - (i * (_Q * _Q) + j * _Q + k)


def _fill_body(qf_hbm, cc_hbm, out_hbm, qf_v, cc_v, row_v, sems):
    wid = lax.axis_index("s") * _NC + lax.axis_index("c")
    pltpu.sync_copy(qf_hbm, qf_v)
    pltpu.sync_copy(cc_hbm, cc_v)

    start = wid * 86 - jnp.maximum(wid - 24, 0)
    count = jnp.where(wid < 24, 86, 85)

    def build_group(gstart, parity):
        bases = [_row_base(gstart + r) for r in range(_G)]

        @plsc.parallel_loop(0, _NCHUNK, unroll=2)
        def chunk_step(t2):
            cc = cc_v[pl.ds(t2 * 16, 16)]
            idxs = [cc + jnp.int32(bases[r]) for r in range(_G)]
            vals = [plsc.load_gather(qf_v, [idxs[r]]) for r in range(_G)]
            for r in range(_G):
                row_v[parity, r, pl.ds(t2 * 16, 16)] = vals[r]

    for g in range(_NGRP):
        parity = g % 2
        gstart = jnp.minimum(start + g * _G, start + count - _G)
        if g >= 2:
            pltpu.make_async_copy(
                row_v.at[parity, :, pl.ds(0, _NROW)],
                out_hbm.at[pl.ds(0, _G)], sems.at[parity]).wait()
        build_group(gstart, parity)
        pltpu.async_copy(
            row_v.at[parity, :, pl.ds(0, _NROW)],
            out_hbm.at[pl.ds(gstart, _G)], sems.at[parity])

    for parity in range(2):
        pltpu.make_async_copy(
            row_v.at[parity, :, pl.ds(0, _NROW)],
            out_hbm.at[pl.ds(0, _G)], sems.at[parity]).wait()


@jax.jit
def _sc_fill(qf, colcode):
    mesh = plsc.VectorSubcoreMesh(
        core_axis_name="c", subcore_axis_name="s",
        num_cores=_NC, num_subcores=_NS,
    )
    return pl.kernel(
        _fill_body,
        out_type=jax.ShapeDtypeStruct((_NROW, _NROW), jnp.float32),
        mesh=mesh,
        scratch_types=[
            pltpu.VMEM((_QFPAD,), jnp.float32),
            pltpu.VMEM((_ROWPAD,), jnp.int32),
            pltpu.VMEM((2, _G, _ROWPAD), jnp.float32),
            pltpu.SemaphoreType.DMA((2,)),
        ],
        compiler_params=pltpu.CompilerParams(
            use_tc_tiling_on_sc=False, needs_layout_passes=False),
    )(qf, colcode)


def kernel(biases, feat_H, feat_W, feat_D):
    del feat_H, feat_W, feat_D
    q = jnp.pad(biases.astype(jnp.float32), _PAD)
    qf = jnp.zeros((_QFPAD,), jnp.float32).at[: _Q ** 3].set(q.reshape(-1))
    abc = jnp.arange(_NROW, dtype=jnp.int32)
    a, b, c = abc // (_N * _N), (abc // _N) % _N, abc % _N
    colcode = jnp.zeros((_ROWPAD,), jnp.int32).at[:_NROW].set(
        a * (_Q * _Q) + b * _Q + c)
    return _sc_fill(qf, colcode)


# speedup vs baseline: 1.0604x; 1.0157x over previous
import functools

import jax
import jax.numpy as jnp
from jax import lax
from jax.experimental import pallas as pl
from jax.experimental.pallas import tpu as pltpu
from jax.experimental.pallas import tpu_sc as plsc

_N = 14
_R = 4
_Q = 2 * _N - 1
_PAD = _N - 1 - _R
_NC = 2
_NS = 16
_NW = _NC * _NS
_NROW = _N * _N * _N
_ROWPAD = ((_NROW + 15) // 16) * 16
_NCHUNK = _ROWPAD // 16
_QFPAD = ((_Q ** 3 + 7) // 8) * 8
_C0 = (_N - 1) * (_Q * _Q + _Q + 1)
_G = 8
_NBUF = 4
_NGRP = 11


def _row_base(q):
    i = q // (_N * _N)
    rem = q - i * (_N * _N)
    j = rem // _N
    k = rem - j * _N
    return _C0 - (i * (_Q * _Q) + j * _Q + k)


def _fill_body(qf_hbm, cc_hbm, out_hbm, qf_v, cc_v, row_v, sems):
    wid = lax.axis_index("s") * _NC + lax.axis_index("c")
    pltpu.sync_copy(qf_hbm, qf_v)
    pltpu.sync_copy(cc_hbm, cc_v)

    start = wid * 86 - jnp.maximum(wid - 24, 0)
    count = jnp.where(wid < 24, 86, 85)

    def build_group(gstart, parity):
        bases = [_row_base(gstart + r) for r in range(_G)]

        @plsc.parallel_loop(0, _NCHUNK, unroll=2)
        def chunk_step(t2):
            cc = cc_v[pl.ds(t2 * 16, 16)]
            idxs = [cc + jnp.int32(bases[r]) for r in range(_G)]
            vals = [plsc.load_gather(qf_v, [idxs[r]]) for r in range(_G)]
            for r in range(_G):
                row_v[parity, r, pl.ds(t2 * 16, 16)] = vals[r]

    for g in range(_NGRP):
        parity = g % _NBUF
        gstart = jnp.minimum(start + g * _G, start + count - _G)
        if g >= _NBUF:
            pltpu.make_async_copy(
                row_v.at[parity, :, pl.ds(0, _NROW)],
                out_hbm.at[pl.ds(0, _G)], sems.at[parity]).wait()
        build_group(gstart, parity)
        pltpu.async_copy(
            row_v.at[parity, :, pl.ds(0, _NROW)],
            out_hbm.at[pl.ds(gstart, _G)], sems.at[parity])

    for parity in range(_NBUF):
        pltpu.make_async_copy(
            row_v.at[parity, :, pl.ds(0, _NROW)],
            out_hbm.at[pl.ds(0, _G)], sems.at[parity]).wait()


@jax.jit
def _sc_fill(qf, colcode):
    mesh = plsc.VectorSubcoreMesh(
        core_axis_name="c", subcore_axis_name="s",
        num_cores=_NC, num_subcores=_NS,
    )
    return pl.kernel(
        _fill_body,
        out_type=jax.ShapeDtypeStruct((_NROW, _NROW), jnp.float32),
        mesh=mesh,
        scratch_types=[
            pltpu.VMEM((_QFPAD,), jnp.float32),
            pltpu.VMEM((_ROWPAD,), jnp.int32),
            pltpu.VMEM((_NBUF, _G, _ROWPAD), jnp.float32),
            pltpu.SemaphoreType.DMA((_NBUF,)),
        ],
        compiler_params=pltpu.CompilerParams(
            use_tc_tiling_on_sc=False, needs_layout_passes=False),
    )(qf, colcode)


def kernel(biases, feat_H, feat_W, feat_D):
    del feat_H, feat_W, feat_D
    q = jnp.pad(biases.astype(jnp.float32), _PAD)
    qf = jnp.zeros((_QFPAD,), jnp.float32).at[: _Q ** 3].set(q.reshape(-1))
    abc = jnp.arange(_NROW, dtype=jnp.int32)
    a, b, c = abc // (_N * _N), (abc // _N) % _N, abc % _N
    colcode = jnp.zeros((_ROWPAD,), jnp.int32).at[:_NROW].set(
        a * (_Q * _Q) + b * _Q + c)
    return _sc_fill(qf, colcode)
